# Initial kernel scaffold; baseline (speedup 1.0000x reference)
import jax
import jax.numpy as jnp
from jax.experimental import pallas as pl


def kernel(boxes, scores):
    raise NotImplementedError("write your pallas kernel here")


# speedup vs baseline: 24.4865x; 24.4865x over previous
import jax
import jax.numpy as jnp
from jax import lax
from jax.experimental import pallas as pl

_N = 20000
_ROWS = 160
_LANES = 128
_PAD = _ROWS * _LANES
_K = 100
_SCORE_THRESH = 0.05
_NMS_THRESH = 0.5


def _nms_body(x1_ref, y1_ref, x2_ref, y2_ref, s_ref, out_ref):
    x1 = x1_ref[...]
    y1 = y1_ref[...]
    x2 = x2_ref[...]
    y2 = y2_ref[...]
    s0 = s_ref[...]

    neg = -jnp.inf
    row = lax.broadcasted_iota(jnp.int32, (_ROWS, _LANES), 0)
    col = lax.broadcasted_iota(jnp.int32, (_ROWS, _LANES), 1)
    lin = row * _LANES + col
    in_range = lin < _N
    s = jnp.where((s0 > _SCORE_THRESH) & in_range, s0, neg)

    area = (x2 - x1) * (y2 - y1)
    coli = lax.broadcasted_iota(jnp.int32, (1, _LANES), 1)
    big = jnp.int32(2 ** 30)

    def body(i, s):
        m = jnp.max(s)
        has = m > neg
        idx = jnp.min(jnp.where(s == m, lin, big))
        sel = lin == idx
        bx1 = jnp.sum(jnp.where(sel, x1, 0.0))
        by1 = jnp.sum(jnp.where(sel, y1, 0.0))
        bx2 = jnp.sum(jnp.where(sel, x2, 0.0))
        by2 = jnp.sum(jnp.where(sel, y2, 0.0))
        barea = (bx2 - bx1) * (by2 - by1)
        ix1 = jnp.maximum(bx1, x1)
        iy1 = jnp.maximum(by1, y1)
        ix2 = jnp.minimum(bx2, x2)
        iy2 = jnp.minimum(by2, y2)
        inter = jnp.maximum(ix2 - ix1, 0.0) * jnp.maximum(iy2 - iy1, 0.0)
        union = barea + area - inter
        iou = inter / jnp.maximum(union, 1e-8)
        suppress = iou > _NMS_THRESH
        s = jnp.where(has & suppress, neg, s)
        s = jnp.where(sel, neg, s)
        vals = jnp.where(coli == 0, bx1,
               jnp.where(coli == 1, by1,
               jnp.where(coli == 2, bx2,
               jnp.where(coli == 3, by2,
               jnp.where(coli == 4, m, 0.0)))))
        out_ref[pl.ds(i, 1), :] = jnp.where(has, vals, 0.0)
        return s

    lax.fori_loop(0, _K, body, s)


def kernel(boxes, scores):
    bt = jnp.transpose(boxes)
    pad = _PAD - _N
    bt = jnp.pad(bt, ((0, 0), (0, pad)))
    x1 = bt[0].reshape(_ROWS, _LANES)
    y1 = bt[1].reshape(_ROWS, _LANES)
    x2 = bt[2].reshape(_ROWS, _LANES)
    y2 = bt[3].reshape(_ROWS, _LANES)
    s = jnp.pad(scores, (0, pad)).reshape(_ROWS, _LANES)

    out = pl.pallas_call(
        _nms_body,
        out_shape=jax.ShapeDtypeStruct((_K, _LANES), jnp.float32),
    )(x1, y1, x2, y2, s)
    return out[:, :5]
